# baseline (speedup 1.0000x reference)
import functools

import jax
import jax.numpy as jnp
from jax import lax
from jax.experimental import pallas as pl
from jax.experimental.pallas import tpu as pltpu
from jax.experimental.pallas import tpu_sc as plsc

NUM_CORES = 2
NUM_SUBCORES = 16
NUM_WORKERS = NUM_CORES * NUM_SUBCORES
CHUNK_B = 8
NBUF = 4


@functools.lru_cache(maxsize=None)
def _make_gather(n_b, h, d):
    n_rows = n_b * h
    b_per_w = n_b // NUM_WORKERS
    n_per_w = b_per_w * h
    chunk = CHUNK_B * h
    n_chunks = b_per_w // CHUNK_B
    assert b_per_w % CHUNK_B == 0 and n_chunks % NBUF == 0 and n_chunks >= 2 * NBUF
    mesh = plsc.VectorSubcoreMesh(
        core_axis_name="c", subcore_axis_name="s",
        num_cores=NUM_CORES, num_subcores=NUM_SUBCORES)

    @functools.partial(
        pl.kernel,
        mesh=mesh,
        out_type=jax.ShapeDtypeStruct((n_b, h, d), jnp.float32),
        scratch_types=[
            pltpu.VMEM((n_per_w,), jnp.int32),
            [pltpu.VMEM((chunk, d), jnp.float32) for _ in range(NBUF)],
            [pltpu.SemaphoreType.DMA for _ in range(NBUF)],
            [pltpu.SemaphoreType.DMA for _ in range(NBUF)],
        ],
        compiler_params=pltpu.CompilerParams(use_tc_tiling_on_sc=False),
    )
    def gather_kernel(table_hbm, idx_hbm, out_hbm, idx_v, rows, gsem, ssem):
        wid = lax.axis_index("s") * NUM_CORES + lax.axis_index("c")
        base = wid * n_per_w
        b_base = wid * b_per_w

        def start_gather(t, b):
            pltpu.async_copy(
                table_hbm.at[idx_v.at[pl.ds(t * chunk, chunk)]],
                rows[b], gsem[b])

        def wait_gather(t, b):
            pltpu.make_async_copy(
                table_hbm.at[idx_v.at[pl.ds(t * chunk, chunk)]],
                rows[b], gsem[b]).wait()

        def start_store(t, b):
            for i in range(CHUNK_B):
                pltpu.async_copy(
                    rows[b].at[pl.ds(i * h, h)],
                    out_hbm.at[b_base + t * CHUNK_B + i], ssem[b])

        def wait_store(t, b):
            for i in range(CHUNK_B):
                pltpu.make_async_copy(
                    rows[b].at[pl.ds(i * h, h)],
                    out_hbm.at[b_base + t * CHUNK_B + i], ssem[b]).wait()

        pltpu.sync_copy(idx_hbm.at[pl.ds(base, n_per_w)], idx_v)

        for b in range(NBUF):
            start_gather(b, b)

        def body(k, carry):
            t0 = k * NBUF
            for b in range(NBUF):
                wait_gather(t0 + b, b)
                start_store(t0 + b, b)
            for b in range(NBUF):
                t = t0 + b + NBUF
                wait_store(t - NBUF, b)
                start_gather(t, b)
            return carry

        lax.fori_loop(0, n_chunks // NBUF - 1, body, 0)

        t0 = n_chunks - NBUF
        for b in range(NBUF):
            wait_gather(t0 + b, b)
            start_store(t0 + b, b)
        for b in range(NBUF):
            wait_store(t0 + b, b)

    return gather_kernel


def kernel(label, bb, table):
    del bb
    b, h = label.shape
    idx = label.reshape(b * h).astype(jnp.int32)
    return _make_gather(b, h, table.shape[1])(table, idx)
